# baseline (speedup 1.0000x reference)
import functools

import jax
import jax.numpy as jnp
from jax import lax
from jax.experimental import pallas as pl
from jax.experimental.pallas import tpu as pltpu
from jax.experimental.pallas import tpu_sc as plsc

NUM_DAYS = 512
EMB = 128
NC, NS = 2, 16
NW = NC * NS
CHUNK = 128
CNT_W = 128
NSLOT = 3
LANES = 16


def _sc_partial_sums(nid2, cid2, did2, news_table, cat_table):
    n_chunks_total = nid2.shape[0]
    n_chunks = n_chunks_total // NW
    rows_per_sub = NUM_DAYS // NS
    mesh = plsc.VectorSubcoreMesh(core_axis_name="c", subcore_axis_name="s")

    @functools.partial(
        pl.kernel,
        out_type=jax.ShapeDtypeStruct((NC, NUM_DAYS, EMB), jnp.float32),
        mesh=mesh,
        scratch_types=[
            pltpu.VMEM((3, n_chunks, CHUNK), jnp.int32),
            pltpu.VMEM((NSLOT, CHUNK, EMB), jnp.float32),
            pltpu.VMEM((NSLOT, CHUNK, EMB), jnp.float32),
            pltpu.VMEM((NUM_DAYS // NS, EMB), jnp.float32),
            pltpu.VMEM_SHARED((NUM_DAYS, EMB), jnp.float32),
        ] + [pltpu.SemaphoreType.DMA] * (2 * NSLOT),
    )
    def k(nid_hbm, cid_hbm, did_hbm, news_hbm, cat_hbm, zp_hbm,
          idx_v, bufn_v, bufc_v, zinit_v, zacc_s, *sems):
        gsem = sems[:NSLOT]
        ssem = sems[NSLOT:]
        core = lax.axis_index("c")
        sid = lax.axis_index("s")
        wid = sid * NC + core
        my_rows = pl.ds(sid * rows_per_sub, rows_per_sub)
        my_chunks = pl.ds(wid * n_chunks, n_chunks)

        pltpu.sync_copy(nid_hbm.at[my_chunks], idx_v.at[0])
        pltpu.sync_copy(cid_hbm.at[my_chunks], idx_v.at[1])
        pltpu.sync_copy(did_hbm.at[my_chunks], idx_v.at[2])

        gd = [None] * NSLOT
        sd = [None] * NSLOT

        def fire_gathers(i):
            s = i % NSLOT
            gd[s] = [
                pltpu.async_copy(news_hbm.at[idx_v.at[0].at[i]],
                                 bufn_v.at[s], gsem[s]),
                pltpu.async_copy(cat_hbm.at[idx_v.at[1].at[i]],
                                 bufc_v.at[s], gsem[s]),
            ]

        def fire_scatters(i):
            s = i % NSLOT
            for d in gd[s]:
                d.wait()

            @pl.loop(0, CHUNK, step=2)
            def _(r):
                for rr in range(2):
                    for c in range(EMB // LANES):
                        sl = pl.ds(c * LANES, LANES)
                        bufn_v[s, r + rr, sl] = (bufn_v[s, r + rr, sl]
                                                 + bufc_v[s, r + rr, sl])

            day_idx = idx_v.at[2].at[i]
            sd[s] = [
                pltpu.async_copy(bufn_v.at[s], zacc_s.at[day_idx],
                                 ssem[s], add=True),
            ]

        fire_gathers(0)


        zero_r = jnp.zeros((LANES,), jnp.float32)

        @pl.loop(0, rows_per_sub)
        def _(r):
            for c in range(EMB // LANES):
                zinit_v[r, pl.ds(c * LANES, LANES)] = zero_r

        pltpu.sync_copy(zinit_v, zacc_s.at[my_rows])

        plsc.subcore_barrier()

        for i in range(1, n_chunks):
            s = i % NSLOT
            if sd[s] is not None:
                for d in sd[s]:
                    d.wait()
                sd[s] = None
            fire_gathers(i)
            fire_scatters(i - 1)
        fire_scatters(n_chunks - 1)
        for slot in range(NSLOT):
            if sd[slot] is not None:
                for d in sd[slot]:
                    d.wait()

        plsc.subcore_barrier()
        pltpu.sync_copy(zacc_s.at[my_rows], zp_hbm.at[core].at[my_rows])

    return k(nid2, cid2, did2, news_table, cat_table)


HB = 2048


def _tc_day_histogram(did2):
    n = did2.size

    def body(ids_ref, out_ref):
        @pl.when(pl.program_id(0) == 0)
        def _():
            out_ref[...] = jnp.zeros_like(out_ref)

        x = ids_ref[...].reshape(HB, 1)
        days = lax.broadcasted_iota(jnp.int32, (1, NUM_DAYS), 1)
        eq = (x == days).astype(jnp.float32)
        out_ref[...] += jnp.sum(eq, axis=0, keepdims=True)

    return pl.pallas_call(
        body,
        grid=(n // HB,),
        in_specs=[pl.BlockSpec((1, 1, HB), lambda i: (i, 0, 0))],
        out_specs=pl.BlockSpec((1, NUM_DAYS), lambda i: (0, 0)),
        out_shape=jax.ShapeDtypeStruct((1, NUM_DAYS), jnp.float32),
    )(did2.reshape(n // HB, 1, HB))


def _tc_combine(zp, counts):
    def body(zp_ref, cnt_ref, out_ref):
        z = zp_ref[0] + zp_ref[1]
        c = cnt_ref[...].reshape(NUM_DAYS, 1)
        out_ref[...] = z / jnp.maximum(c, 1.0)

    return pl.pallas_call(
        body,
        out_shape=jax.ShapeDtypeStruct((NUM_DAYS, EMB), jnp.float32),
    )(zp, counts)


def kernel(news_ids, category_ids, day_ids, delta_days, news_table, cat_table):
    n = news_ids.shape[0]
    n_chunks_total = n // CHUNK
    nid2 = news_ids.astype(jnp.int32).reshape(n_chunks_total, CHUNK)
    cid2 = category_ids.astype(jnp.int32).reshape(n_chunks_total, CHUNK)
    did2 = day_ids.astype(jnp.int32).reshape(n_chunks_total, CHUNK)
    counts = _tc_day_histogram(did2)
    zp = _sc_partial_sums(nid2, cid2, did2, news_table, cat_table)
    Z = _tc_combine(zp, counts)
    return (Z, delta_days.astype(jnp.float32))


# speedup vs baseline: 1.0149x; 1.0149x over previous
import functools

import jax
import jax.numpy as jnp
from jax import lax
from jax.experimental import pallas as pl
from jax.experimental.pallas import tpu as pltpu
from jax.experimental.pallas import tpu_sc as plsc

NUM_DAYS = 512
EMB = 128
NC, NS = 2, 16
NW = NC * NS
CHUNK = 128
CNT_W = 128
NSLOT = 3
LANES = 16


def _sc_partial_sums(nid2, cid2, did2, news_table, cat_table):
    n_chunks_total = nid2.shape[0]
    n_chunks = n_chunks_total // NW
    rows_per_sub = NUM_DAYS // NS
    mesh = plsc.VectorSubcoreMesh(core_axis_name="c", subcore_axis_name="s")

    @functools.partial(
        pl.kernel,
        out_type=jax.ShapeDtypeStruct((NC, NUM_DAYS, EMB), jnp.float32),
        mesh=mesh,
        scratch_types=[
            pltpu.VMEM((3, n_chunks, CHUNK), jnp.int32),
            pltpu.VMEM((NSLOT, CHUNK, EMB), jnp.float32),
            pltpu.VMEM((NSLOT, CHUNK, EMB), jnp.float32),
            pltpu.VMEM((NUM_DAYS // NS, EMB), jnp.float32),
            pltpu.VMEM_SHARED((NUM_DAYS, EMB), jnp.float32),
        ] + [pltpu.SemaphoreType.DMA] * (2 * NSLOT),
    )
    def k(nid_hbm, cid_hbm, did_hbm, news_hbm, cat_hbm, zp_hbm,
          idx_v, bufn_v, bufc_v, zinit_v, zacc_s, *sems):
        gsem = sems[:NSLOT]
        ssem = sems[NSLOT:]
        core = lax.axis_index("c")
        sid = lax.axis_index("s")
        wid = sid * NC + core
        my_rows = pl.ds(sid * rows_per_sub, rows_per_sub)
        my_chunks = pl.ds(wid * n_chunks, n_chunks)

        pltpu.sync_copy(nid_hbm.at[my_chunks], idx_v.at[0])
        pltpu.sync_copy(cid_hbm.at[my_chunks], idx_v.at[1])
        pltpu.sync_copy(did_hbm.at[my_chunks], idx_v.at[2])

        gd = [None] * NSLOT
        sd = [None] * NSLOT

        def fire_gathers(i):
            s = i % NSLOT
            gd[s] = [
                pltpu.async_copy(news_hbm.at[idx_v.at[0].at[i]],
                                 bufn_v.at[s], gsem[s]),
                pltpu.async_copy(cat_hbm.at[idx_v.at[1].at[i]],
                                 bufc_v.at[s], gsem[s]),
            ]

        def fire_scatters(i):
            s = i % NSLOT
            for d in gd[s]:
                d.wait()

            @pl.loop(0, CHUNK)
            def _(r):
                for c in range(EMB // LANES):
                    sl = pl.ds(c * LANES, LANES)
                    bufn_v[s, r, sl] = bufn_v[s, r, sl] + bufc_v[s, r, sl]

            day_idx = idx_v.at[2].at[i]
            sd[s] = [
                pltpu.async_copy(bufn_v.at[s], zacc_s.at[day_idx],
                                 ssem[s], add=True),
            ]

        fire_gathers(0)


        zero_r = jnp.zeros((LANES,), jnp.float32)

        @pl.loop(0, rows_per_sub)
        def _(r):
            for c in range(EMB // LANES):
                zinit_v[r, pl.ds(c * LANES, LANES)] = zero_r

        pltpu.sync_copy(zinit_v, zacc_s.at[my_rows])

        plsc.subcore_barrier()

        for i in range(1, n_chunks):
            s = i % NSLOT
            if sd[s] is not None:
                for d in sd[s]:
                    d.wait()
                sd[s] = None
            fire_gathers(i)
            fire_scatters(i - 1)
        fire_scatters(n_chunks - 1)
        for slot in range(NSLOT):
            if sd[slot] is not None:
                for d in sd[slot]:
                    d.wait()

        plsc.subcore_barrier()
        pltpu.sync_copy(zacc_s.at[my_rows], zp_hbm.at[core].at[my_rows])

    return k(nid2, cid2, did2, news_table, cat_table)


HB = 2048


def _tc_day_histogram(did2):
    n = did2.size

    def body(ids_ref, out_ref):
        @pl.when(pl.program_id(0) == 0)
        def _():
            out_ref[...] = jnp.zeros_like(out_ref)

        x = ids_ref[...].reshape(HB, 1)
        days = lax.broadcasted_iota(jnp.int32, (1, NUM_DAYS), 1)
        eq = (x == days).astype(jnp.float32)
        out_ref[...] += jnp.sum(eq, axis=0, keepdims=True)

    return pl.pallas_call(
        body,
        grid=(n // HB,),
        in_specs=[pl.BlockSpec((1, 1, HB), lambda i: (i, 0, 0))],
        out_specs=pl.BlockSpec((1, NUM_DAYS), lambda i: (0, 0)),
        out_shape=jax.ShapeDtypeStruct((1, NUM_DAYS), jnp.float32),
    )(did2.reshape(n // HB, 1, HB))


def _tc_combine(zp, counts):
    def body(zp_ref, cnt_ref, out_ref):
        z = zp_ref[0] + zp_ref[1]
        c = cnt_ref[...].reshape(NUM_DAYS, 1)
        out_ref[...] = z / jnp.maximum(c, 1.0)

    return pl.pallas_call(
        body,
        out_shape=jax.ShapeDtypeStruct((NUM_DAYS, EMB), jnp.float32),
    )(zp, counts)


def kernel(news_ids, category_ids, day_ids, delta_days, news_table, cat_table):
    n = news_ids.shape[0]
    n_chunks_total = n // CHUNK
    nid2 = news_ids.astype(jnp.int32).reshape(n_chunks_total, CHUNK)
    cid2 = category_ids.astype(jnp.int32).reshape(n_chunks_total, CHUNK)
    did2 = day_ids.astype(jnp.int32).reshape(n_chunks_total, CHUNK)
    counts = _tc_day_histogram(did2)
    zp = _sc_partial_sums(nid2, cid2, did2, news_table, cat_table)
    Z = _tc_combine(zp, counts)
    return (Z, delta_days.astype(jnp.float32))
